# baseline (speedup 1.0000x reference)
import jax
import jax.numpy as jnp
from jax import lax
from jax.experimental import pallas as pl
from jax.experimental.pallas import tpu as pltpu
from jax.experimental.pallas import tpu_sc as plsc

VOCAB = 100000
HIDDEN = 64
B = 4096
C = 50
T = 50

NC = 2
NS = 16
NW = NC * NS
BPW = B // NW
NP = BPW // 2

TBLW = 2 * HIDDEN
TBLR = VOCAB // 2

CP = 64
TP = 64
TG = 4
TPAD = TG * 16
IW = 2 * CP
ROWS = IW


def _cbow_body(chi_hbm, thi_hbm, csel_hbm, tsel_hbm, in_emb_hbm, out_emb_hbm,
               y_hbm,
               chi_v, thi_v, csel_v, tsel_v, cr_a, cr_b, tr_a, tr_b, out_v,
               sem_i, sem_ca, sem_cb, sem_ta, sem_tb, sem_out):
    wid = lax.axis_index("s") * NC + lax.axis_index("c")
    base = wid * BPW
    lane = lax.iota(jnp.int32, 16)

    pltpu.async_copy(chi_hbm.at[pl.ds(wid * NP, NP)], chi_v, sem_i)
    pltpu.async_copy(thi_hbm.at[pl.ds(wid * NP, NP)], thi_v, sem_i)
    pltpu.async_copy(csel_hbm.at[pl.ds(wid * NP, NP)], csel_v, sem_i)
    pltpu.async_copy(tsel_hbm.at[pl.ds(wid * NP, NP)], tsel_v, sem_i)
    pltpu.make_async_copy(chi_hbm.at[pl.ds(wid * NP, NP)], chi_v, sem_i).wait()
    pltpu.make_async_copy(thi_hbm.at[pl.ds(wid * NP, NP)], thi_v, sem_i).wait()
    pltpu.make_async_copy(csel_hbm.at[pl.ds(wid * NP, NP)], csel_v, sem_i).wait()
    pltpu.make_async_copy(tsel_hbm.at[pl.ds(wid * NP, NP)], tsel_v, sem_i).wait()

    def fire(p, crows, trows, sem_c, sem_t):
        pltpu.async_copy(in_emb_hbm.at[chi_v.at[p]], crows, sem_c)
        pltpu.async_copy(out_emb_hbm.at[thi_v.at[p]], trows, sem_t)

    def drain(p, crows, trows, sem_c, sem_t):
        pltpu.make_async_copy(in_emb_hbm.at[chi_v.at[p]], crows, sem_c).wait()
        pltpu.make_async_copy(out_emb_hbm.at[thi_v.at[p]], trows, sem_t).wait()

    def compute_one(p, i, crows, trows, cbase, tbase):
        z = jnp.zeros((16,), jnp.float32)
        a0, a1, a2, a3 = z, z, z, z
        for g in range(4):
            kmax = min(16, C - g * 16)
            svec = csel_v[p, pl.ds(cbase + g * 16, 16)]
            for k in range(kmax):
                cc = cbase + g * 16 + k
                s = svec[k]
                a0 = a0 + crows[cc, pl.ds(s, 16)]
                a1 = a1 + crows[cc, pl.ds(s + 16, 16)]
                a2 = a2 + crows[cc, pl.ds(s + 32, 16)]
                a3 = a3 + crows[cc, pl.ds(s + 48, 16)]
        con0, con1, con2, con3 = a0, a1, a2, a3

        def sgroup(g, carry2):
            tb = tbase + g * 16
            acc = jnp.zeros((16,), jnp.float32)
            svec = tsel_v[p, pl.ds(tb, 16)]
            for k in range(16):
                tt = tb + k
                s = svec[k]
                q = trows[tt, pl.ds(s, 16)] * con0
                q = q + trows[tt, pl.ds(s + 16, 16)] * con1
                q = q + trows[tt, pl.ds(s + 32, 16)] * con2
                q = q + trows[tt, pl.ds(s + 48, 16)] * con3
                for sh in (8, 4, 2, 1):
                    q = q + q.at[lane ^ sh].get(mode="promise_in_bounds")
                acc = jnp.where(lane == k, q, acc)
            out_v[i, pl.ds(g * 16, 16)] = acc
            return carry2

        lax.fori_loop(0, TG, sgroup, 0)

    def compute_pair(p, crows, trows):
        compute_one(p, 2 * p, crows, trows, 0, 0)
        compute_one(p, 2 * p + 1, crows, trows, CP, TP)

    fire(0, cr_a, tr_a, sem_ca, sem_ta)

    def step(j, carry):
        p0 = 2 * j
        fire(p0 + 1, cr_b, tr_b, sem_cb, sem_tb)
        drain(p0, cr_a, tr_a, sem_ca, sem_ta)
        compute_pair(p0, cr_a, tr_a)
        pnext = jnp.minimum(p0 + 2, NP - 1)
        fire(pnext, cr_a, tr_a, sem_ca, sem_ta)
        drain(p0 + 1, cr_b, tr_b, sem_cb, sem_tb)
        compute_pair(p0 + 1, cr_b, tr_b)
        return carry

    lax.fori_loop(0, NP // 2, step, 0)
    drain(NP - 1, cr_a, tr_a, sem_ca, sem_ta)

    pltpu.async_copy(out_v, y_hbm.at[pl.ds(base, BPW)], sem_out)
    pltpu.make_async_copy(out_v, y_hbm.at[pl.ds(base, BPW)], sem_out).wait()


@jax.jit
def _cbow_sc(chi, thi, csel, tsel, in_emb2, out_emb2):
    mesh = plsc.VectorSubcoreMesh(core_axis_name="c", subcore_axis_name="s")
    f = pl.kernel(
        _cbow_body,
        out_type=jax.ShapeDtypeStruct((B, TPAD), jnp.float32),
        mesh=mesh,
        scratch_types=[
            pltpu.VMEM((NP, IW), jnp.int32),
            pltpu.VMEM((NP, IW), jnp.int32),
            pltpu.VMEM((NP, IW), jnp.int32),
            pltpu.VMEM((NP, IW), jnp.int32),
            pltpu.VMEM((ROWS, TBLW), jnp.float32),
            pltpu.VMEM((ROWS, TBLW), jnp.float32),
            pltpu.VMEM((ROWS, TBLW), jnp.float32),
            pltpu.VMEM((ROWS, TBLW), jnp.float32),
            pltpu.VMEM((BPW, TPAD), jnp.float32),
            pltpu.SemaphoreType.DMA,
            pltpu.SemaphoreType.DMA,
            pltpu.SemaphoreType.DMA,
            pltpu.SemaphoreType.DMA,
            pltpu.SemaphoreType.DMA,
            pltpu.SemaphoreType.DMA,
        ],
        compiler_params=pltpu.CompilerParams(use_tc_tiling_on_sc=True),
    )
    return f(chi, thi, csel, tsel, in_emb2, out_emb2)


def kernel(contexts, t, in_emb, out_emb):
    contexts = contexts.astype(jnp.int32)
    t = t.astype(jnp.int32)
    ctx_pad = jnp.pad(contexts, ((0, 0), (0, CP - C)))
    tid_pad = jnp.pad(t, ((0, 0), (0, TP - T)))
    chi = (ctx_pad >> 1).reshape(B // 2, IW)
    thi = (tid_pad >> 1).reshape(B // 2, IW)
    csel = ((ctx_pad & 1) << 6).reshape(B // 2, IW)
    tsel = ((tid_pad & 1) << 6).reshape(B // 2, IW)
    in_emb2 = in_emb.reshape(TBLR, TBLW)
    out_emb2 = out_emb.reshape(TBLR, TBLW)
    y = _cbow_sc(chi, thi, csel, tsel, in_emb2, out_emb2)
    return y[:, :T].reshape(B, 1, T)
